# baseline (speedup 1.0000x reference)
import jax
import jax.numpy as jnp
from jax.experimental import pallas as pl
from jax.experimental.pallas import tpu as pltpu


def _body(H, NB, inv_n, thr_ref, mask_ref, probs_ref, scores_ref, newmask_ref):
    h = pl.program_id(1)
    nb = pl.program_id(2)
    w = (mask_ref[0, :, 0] >= 0).astype(jnp.float32)
    part = jnp.dot(w[None, :], probs_ref[0, 0],
                   preferred_element_type=jnp.float32)[None]
    first = jnp.logical_and(h == 0, nb == 0)

    @pl.when(first)
    def _():
        scores_ref[...] = part

    @pl.when(jnp.logical_not(first))
    def _():
        scores_ref[...] += part

    @pl.when(jnp.logical_and(h == H - 1, nb == NB - 1))
    def _():
        s = scores_ref[...] * inv_n
        scores_ref[...] = s
        newmask_ref[...] = jnp.where(s < thr_ref[0, 0], -10000.0, 0.0)


def kernel(attention_mask, attention_probs, sentence_lengths, keep_threshold):
    B, H, S, _ = attention_probs.shape
    BS = 512
    NB = S // BS
    mask3 = attention_mask.reshape(B, S, 1)
    thr = jnp.maximum(jnp.float32(1e-5), keep_threshold).reshape(1, 1)

    import functools
    body = functools.partial(_body, H, NB, 1.0 / (H * S))

    scores, newmask = pl.pallas_call(
        body,
        grid=(B, H, NB),
        in_specs=[
            pl.BlockSpec(memory_space=pltpu.SMEM),
            pl.BlockSpec((1, BS, 1), lambda b, h, nb: (b, nb, 0)),
            pl.BlockSpec((1, 1, BS, S), lambda b, h, nb: (b, h, nb, 0)),
        ],
        out_specs=[
            pl.BlockSpec((1, 1, S), lambda b, h, nb: (b, 0, 0)),
            pl.BlockSpec((1, 1, S), lambda b, h, nb: (b, 0, 0)),
        ],
        out_shape=[
            jax.ShapeDtypeStruct((B, 1, S), jnp.float32),
            jax.ShapeDtypeStruct((B, 1, S), jnp.float32),
        ],
        compiler_params=pltpu.CompilerParams(
            dimension_semantics=("arbitrary", "arbitrary", "arbitrary"),
        ),
    )(thr, mask3, attention_probs)

    return (newmask.reshape(B, 1, 1, S), keep_threshold, scores.reshape(B, S))
